# Initial kernel scaffold; baseline (speedup 1.0000x reference)
import jax
import jax.numpy as jnp
from jax.experimental import pallas as pl


def kernel(embeddings, labels, centers):
    raise NotImplementedError("write your pallas kernel here")


# speedup vs baseline: 8.3644x; 8.3644x over previous
import functools

import jax
import jax.numpy as jnp
from jax.experimental import pallas as pl
from jax.experimental.pallas import tpu as pltpu

_N = 2048
_F = 512
_C = 288
_K = 10
_M1 = 0.3
_M2 = 0.3
_CW = 0.01
_INF = 1e30
_BLK = 256
_NBLK = _N // _BLK


def _cumsum_lanes(x):
    n = x.shape[-1]
    s = 1
    while s < n:
        shifted = jnp.concatenate(
            [jnp.zeros(x.shape[:-1] + (s,), x.dtype), x[..., : n - s]], axis=-1
        )
        x = x + shifted
        s *= 2
    return x


def _loss_kernel(e_ref, lab_r_ref, lab_c_ref, u_ref, cen_ref, out_ref,
                 dist_ref, min1_ref, min2_ref, cstar_ref):
    f32 = jnp.float32
    i32 = jnp.int32
    hp = jax.lax.Precision.HIGHEST

    ee = e_ref[:, :] * e_ref[:, :]
    sq_row = jax.lax.dot_general(
        jnp.ones((1, _F), f32), ee, (((1,), (1,)), ((), ())), precision=hp)

    def phase_a(blk, _):
        rows = blk * _BLK
        eb = e_ref[pl.ds(rows, _BLK), :]
        g = jax.lax.dot_general(
            eb, e_ref[:, :], (((1,), (1,)), ((), ())), precision=hp)
        sq_b = jnp.sum(eb * eb, axis=1, keepdims=True)
        d2 = jnp.maximum(sq_b + sq_row - 2.0 * g, 0.0)
        dist_ref[pl.ds(rows, _BLK), :] = jnp.where(d2 > 0.0, jnp.sqrt(d2), 0.0)
        return 0

    jax.lax.fori_loop(0, _NBLK, phase_a, 0)

    lab_c = lab_c_ref[:, :]

    def phase_a2(blk, _):
        cols = blk * _BLK
        slab = dist_ref[:, pl.ds(cols, _BLK)]
        rowi = jax.lax.broadcasted_iota(i32, (_N, _BLK), 0)
        coli = jax.lax.broadcasted_iota(i32, (_N, _BLK), 1) + cols
        dp = jnp.where(rowi == coli, _INF, slab)
        m1 = jnp.min(dp, axis=0, keepdims=True)
        am = jnp.min(jnp.where(dp == m1, rowi, _N), axis=0, keepdims=True)
        ohm = rowi == am
        cst = jnp.sum(
            jnp.where(ohm, jnp.broadcast_to(lab_c, (_N, _BLK)), 0),
            axis=0, keepdims=True)
        m2 = jnp.min(jnp.where(lab_c != cst, dp, _INF), axis=0, keepdims=True)
        min1_ref[0:1, pl.ds(cols, _BLK)] = m1
        min2_ref[0:1, pl.ds(cols, _BLK)] = m2
        cstar_ref[0:1, pl.ds(cols, _BLK)] = cst
        return 0

    jax.lax.fori_loop(0, _NBLK, phase_a2, 0)

    lab_r = lab_r_ref[0:1, :]
    min1_row = min1_ref[0:1, :]
    min2_row = min2_ref[0:1, :]
    cstar_row = cstar_ref[0:1, :]

    def phase_b(blk, carry):
        acc, cnt, csum = carry
        rows = blk * _BLK
        eb = e_ref[pl.ds(rows, _BLK), :]
        db = dist_ref[pl.ds(rows, _BLK), :]
        lb = lab_c_ref[pl.ds(rows, _BLK), :]
        same = lb == lab_r
        col = jax.lax.broadcasted_iota(i32, (_BLK, _N), 1)
        rowid = jax.lax.broadcasted_iota(i32, (_BLK, _N), 0) + rows
        pos_mask = same & (col != rowid)
        num_pos = jnp.sum(pos_mask.astype(i32), axis=1, keepdims=True)
        num_neg = _N - num_pos - 1
        valid = (num_pos > 0) & (num_neg >= 2)

        ub = u_ref[pl.ds(rows, _BLK), :]
        r = jnp.minimum(
            (ub * jnp.maximum(num_pos, 1).astype(f32)).astype(i32),
            jnp.maximum(num_pos - 1, 0))
        order = _cumsum_lanes(pos_mask.astype(i32)) - 1
        hit = pos_mask & (order == r)
        d_ap = jnp.sum(jnp.where(hit, db, 0.0), axis=1, keepdims=True)

        k_eff = jnp.minimum(_K, num_neg)
        work = jnp.where(same, _INF, db)
        for jj in range(_K):
            mn = jnp.min(work, axis=1, keepdims=True)
            am = jnp.min(jnp.where(work == mn, col, _N), axis=1, keepdims=True)
            oh = col == am
            v1g = jnp.sum(jnp.where(oh, jnp.broadcast_to(min1_row, (_BLK, _N)), 0.0),
                          axis=1, keepdims=True)
            v2g = jnp.sum(jnp.where(oh, jnp.broadcast_to(min2_row, (_BLK, _N)), 0.0),
                          axis=1, keepdims=True)
            cg = jnp.sum(jnp.where(oh, jnp.broadcast_to(cstar_row, (_BLK, _N)), 0),
                         axis=1, keepdims=True)
            d_n1n2 = jnp.where(cg != lb, v1g, v2g)
            t1 = jnp.maximum(d_ap - mn + _M1, 0.0)
            t2 = jnp.maximum(d_ap - d_n1n2 + _M2, 0.0)
            vk = valid & (jj < k_eff)
            acc = acc + jnp.sum(jnp.where(vk, t1 + t2, 0.0), keepdims=True)
            cnt = cnt + jnp.sum(vk.astype(i32), keepdims=True)
            if jj < _K - 1:
                work = jnp.where(oh, _INF, work)

        oh_c = (lb == jax.lax.broadcasted_iota(i32, (_BLK, _C), 1)).astype(f32)
        gath = jax.lax.dot_general(
            oh_c, cen_ref[:, :], (((1,), (0,)), ((), ())), precision=hp)
        diff = eb - gath
        csum = csum + jnp.sum(diff * diff, keepdims=True)
        return acc, cnt, csum

    zero = jnp.zeros((1, 1), f32)
    acc, cnt, csum = jax.lax.fori_loop(
        0, _NBLK, phase_b, (zero, jnp.zeros((1, 1), i32), zero))

    cntf = cnt.astype(f32)
    quad = jnp.where(cnt > 0, acc / jnp.maximum(cntf, 1.0), 0.0)
    out_ref[:, :] = quad + _CW * (csum / float(_N))


@jax.jit
def kernel(embeddings, labels, centers):
    labels = labels.astype(jnp.int32)
    u = jax.random.uniform(jax.random.key(42), (_N,))
    out = pl.pallas_call(
        _loss_kernel,
        out_shape=jax.ShapeDtypeStruct((1, 1), jnp.float32),
        scratch_shapes=[
            pltpu.VMEM((_N, _N), jnp.float32),
            pltpu.VMEM((1, _N), jnp.float32),
            pltpu.VMEM((1, _N), jnp.float32),
            pltpu.VMEM((1, _N), jnp.int32),
        ],
    )(embeddings, labels.reshape(1, _N), labels.reshape(_N, 1),
      u.reshape(_N, 1), centers)
    return out[0, 0]


# speedup vs baseline: 12.5947x; 1.5057x over previous
import jax
import jax.numpy as jnp
from jax.experimental import pallas as pl
from jax.experimental.pallas import tpu as pltpu

_N = 2048
_F = 512
_C = 288
_K = 10
_M1 = 0.3
_M2 = 0.3
_CW = 0.01
_INF = 1e30
_BLK = 256
_NBLK = _N // _BLK
_MAXKEY = 2147483647


def _loss_kernel(e_ref, lab_r_ref, lab_c_ref, u_ref, cen_ref, out_ref,
                 dist_ref, tri_ref, min1_ref, min2_ref, cstar_ref):
    f32 = jnp.float32
    i32 = jnp.int32
    hp = jax.lax.Precision.HIGHEST

    ti = jax.lax.broadcasted_iota(i32, (_N, _N), 0)
    tj = jax.lax.broadcasted_iota(i32, (_N, _N), 1)
    tri_ref[:, :] = jnp.where(ti <= tj, 1.0, 0.0).astype(jnp.bfloat16)

    ee = e_ref[:, :] * e_ref[:, :]
    sq_row = jax.lax.dot_general(
        jnp.ones((1, _F), f32), ee, (((1,), (1,)), ((), ())), precision=hp)

    def phase_a(blk, _):
        rows = blk * _BLK
        eb = e_ref[pl.ds(rows, _BLK), :]
        g = jax.lax.dot_general(
            eb, e_ref[:, :], (((1,), (1,)), ((), ())), precision=hp)
        sq_b = jnp.sum(eb * eb, axis=1, keepdims=True)
        d2 = jnp.maximum(sq_b + sq_row - 2.0 * g, 0.0)
        dist_ref[pl.ds(rows, _BLK), :] = jnp.sqrt(d2)
        return 0

    jax.lax.fori_loop(0, _NBLK, phase_a, 0)

    lab_c = lab_c_ref[:, :]

    def phase_a2(blk, _):
        cols = blk * _BLK
        slab = dist_ref[:, pl.ds(cols, _BLK)]
        rowi = jax.lax.broadcasted_iota(i32, (_N, _BLK), 0)
        coli = jax.lax.broadcasted_iota(i32, (_N, _BLK), 1) + cols
        dp = jnp.where(rowi == coli, _INF, slab)
        m1 = jnp.min(dp, axis=0, keepdims=True)
        am = jnp.min(jnp.where(dp == m1, rowi, _N), axis=0, keepdims=True)
        ohm = rowi == am
        cst = jnp.sum(
            jnp.where(ohm, jnp.broadcast_to(lab_c, (_N, _BLK)), 0),
            axis=0, keepdims=True)
        m2 = jnp.min(jnp.where(lab_c != cst, dp, _INF), axis=0, keepdims=True)
        min1_ref[0:1, pl.ds(cols, _BLK)] = m1
        min2_ref[0:1, pl.ds(cols, _BLK)] = m2
        cstar_ref[0:1, pl.ds(cols, _BLK)] = cst
        return 0

    jax.lax.fori_loop(0, _NBLK, phase_a2, 0)

    lab_r = lab_r_ref[0:1, :]
    min1_row = min1_ref[0:1, :]
    min2_row = min2_ref[0:1, :]
    cstar_row = cstar_ref[0:1, :]

    def phase_b(blk, carry):
        acc, cnt, csum = carry
        rows = blk * _BLK
        eb = e_ref[pl.ds(rows, _BLK), :]
        db = dist_ref[pl.ds(rows, _BLK), :]
        lb = lab_c_ref[pl.ds(rows, _BLK), :]
        same = lb == lab_r
        col = jax.lax.broadcasted_iota(i32, (_BLK, _N), 1)
        rowid = jax.lax.broadcasted_iota(i32, (_BLK, _N), 0) + rows
        pos_mask = same & (col != rowid)
        num_pos = jnp.sum(pos_mask.astype(i32), axis=1, keepdims=True)
        num_neg = _N - num_pos - 1
        valid = (num_pos > 0) & (num_neg >= 2)

        ub = u_ref[pl.ds(rows, _BLK), :]
        r = jnp.minimum(
            (ub * jnp.maximum(num_pos, 1).astype(f32)).astype(i32),
            jnp.maximum(num_pos - 1, 0))
        pmf = jnp.where(pos_mask, 1.0, 0.0).astype(jnp.bfloat16)
        order = jax.lax.dot_general(
            pmf, tri_ref[:, :], (((1,), (0,)), ((), ())),
            preferred_element_type=f32)
        hit = pos_mask & (order == (r + 1).astype(f32))
        d_ap = jnp.sum(jnp.where(hit, db, 0.0), axis=1, keepdims=True)

        negd = jnp.where(same, _INF, db)
        bits = jax.lax.bitcast_convert_type(negd, i32)
        keys = (bits & (-2048)) | col
        ohacc = jnp.zeros((_BLK, _N), jnp.bool_)
        for jj in range(_K):
            mnk = jnp.min(keys, axis=1, keepdims=True)
            oh = keys == mnk
            ohacc = ohacc | oh
            if jj < _K - 1:
                keys = jnp.where(oh, _MAXKEY, keys)

        dnn = jnp.where(cstar_row != lb, jnp.broadcast_to(min1_row, (_BLK, _N)),
                        jnp.broadcast_to(min2_row, (_BLK, _N)))
        sel = ohacc & (negd < _INF) & valid
        tt = (jnp.maximum(d_ap + _M1 - negd, 0.0)
              + jnp.maximum(d_ap + _M2 - dnn, 0.0))
        acc = acc + jnp.sum(jnp.where(sel, tt, 0.0), keepdims=True)
        cnt = cnt + jnp.sum(sel.astype(i32), keepdims=True)

        oh_c = (lb == jax.lax.broadcasted_iota(i32, (_BLK, _C), 1)).astype(f32)
        gath = jax.lax.dot_general(
            oh_c, cen_ref[:, :], (((1,), (0,)), ((), ())), precision=hp)
        diff = eb - gath
        csum = csum + jnp.sum(diff * diff, keepdims=True)
        return acc, cnt, csum

    zero = jnp.zeros((1, 1), f32)
    acc, cnt, csum = jax.lax.fori_loop(
        0, _NBLK, phase_b, (zero, jnp.zeros((1, 1), i32), zero))

    cntf = cnt.astype(f32)
    quad = jnp.where(cnt > 0, acc / jnp.maximum(cntf, 1.0), 0.0)
    out_ref[:, :] = quad + _CW * (csum / float(_N))


@jax.jit
def kernel(embeddings, labels, centers):
    labels = labels.astype(jnp.int32)
    u = jax.random.uniform(jax.random.key(42), (_N,))
    out = pl.pallas_call(
        _loss_kernel,
        out_shape=jax.ShapeDtypeStruct((1, 1), jnp.float32),
        scratch_shapes=[
            pltpu.VMEM((_N, _N), jnp.float32),
            pltpu.VMEM((_N, _N), jnp.bfloat16),
            pltpu.VMEM((1, _N), jnp.float32),
            pltpu.VMEM((1, _N), jnp.float32),
            pltpu.VMEM((1, _N), jnp.int32),
        ],
    )(embeddings, labels.reshape(1, _N), labels.reshape(_N, 1),
      u.reshape(_N, 1), centers)
    return out[0, 0]


# speedup vs baseline: 15.5704x; 1.2363x over previous
import jax
import jax.numpy as jnp
from jax.experimental import pallas as pl
from jax.experimental.pallas import tpu as pltpu

_N = 2048
_F = 512
_C = 288
_K = 10
_M1 = 0.3
_M2 = 0.3
_CW = 0.01
_INF = 1e30
_BLK = 256
_NBLK = _N // _BLK
_MAXKEY = 2147483647


def _loss_kernel(e_ref, lab_r_ref, lab_c_ref, u_ref, cen_ref, out_ref,
                 dist_ref, min1_ref, min2_ref, cstar_ref, rank_col_ref):
    f32 = jnp.float32
    i32 = jnp.int32
    bf16 = jnp.bfloat16
    hp = jax.lax.Precision.HIGHEST

    e_all = e_ref[:, :]
    e_hi = e_all.astype(bf16)
    e_lo = (e_all - e_hi.astype(f32)).astype(bf16)
    cen = cen_ref[:, :]
    cen_hi = cen.astype(bf16)
    cen_lo = (cen - cen_hi.astype(f32)).astype(bf16)

    sq_row = jax.lax.dot_general(
        jnp.ones((1, _F), f32), e_all * e_all, (((1,), (1,)), ((), ())),
        precision=hp)

    lab_r = lab_r_ref[0:1, :]
    lab_c = lab_c_ref[:, :]
    ti = jax.lax.broadcasted_iota(i32, (_N, _N), 0)
    tj = jax.lax.broadcasted_iota(i32, (_N, _N), 1)
    same_all = lab_c == lab_r
    rank_row = jnp.sum((same_all & (ti < tj)).astype(i32), axis=0,
                       keepdims=True)
    rank_col_ref[:, :] = jnp.sum((same_all & (tj < ti)).astype(i32), axis=1,
                                 keepdims=True)

    def phase_a(blk, _):
        rows = blk * _BLK
        eb = e_ref[pl.ds(rows, _BLK), :]
        hi_b = eb.astype(bf16)
        lo_b = (eb - hi_b.astype(f32)).astype(bf16)
        dn = (((1,), (1,)), ((), ()))
        g = (jax.lax.dot_general(hi_b, e_hi, dn, preferred_element_type=f32)
             + jax.lax.dot_general(hi_b, e_lo, dn, preferred_element_type=f32)
             + jax.lax.dot_general(lo_b, e_hi, dn, preferred_element_type=f32))
        sq_b = jnp.sum(eb * eb, axis=1, keepdims=True)
        d2 = jnp.maximum(sq_b + sq_row - 2.0 * g, 0.0)
        dist_ref[pl.ds(rows, _BLK), :] = jnp.sqrt(d2)
        return 0

    jax.lax.fori_loop(0, _NBLK, phase_a, 0)

    def phase_a2(blk, _):
        cols = blk * _BLK
        slab = dist_ref[:, pl.ds(cols, _BLK)]
        rowi = jax.lax.broadcasted_iota(i32, (_N, _BLK), 0)
        coli = jax.lax.broadcasted_iota(i32, (_N, _BLK), 1) + cols
        dp = jnp.where(rowi == coli, _INF, slab)
        m1 = jnp.min(dp, axis=0, keepdims=True)
        am = jnp.min(jnp.where(dp == m1, rowi, _N), axis=0, keepdims=True)
        ohm = rowi == am
        cst = jnp.sum(
            jnp.where(ohm, jnp.broadcast_to(lab_c, (_N, _BLK)), 0),
            axis=0, keepdims=True)
        m2 = jnp.min(jnp.where(lab_c != cst, dp, _INF), axis=0, keepdims=True)
        min1_ref[0:1, pl.ds(cols, _BLK)] = m1
        min2_ref[0:1, pl.ds(cols, _BLK)] = m2
        cstar_ref[0:1, pl.ds(cols, _BLK)] = cst
        return 0

    jax.lax.fori_loop(0, _NBLK, phase_a2, 0)

    min1_row = min1_ref[0:1, :]
    min2_row = min2_ref[0:1, :]
    cstar_row = cstar_ref[0:1, :]

    def phase_b(blk, carry):
        acc, cnt, csum = carry
        rows = blk * _BLK
        eb = e_ref[pl.ds(rows, _BLK), :]
        db = dist_ref[pl.ds(rows, _BLK), :]
        lb = lab_c_ref[pl.ds(rows, _BLK), :]
        same = lb == lab_r
        col = jax.lax.broadcasted_iota(i32, (_BLK, _N), 1)
        rowid = jax.lax.broadcasted_iota(i32, (_BLK, _N), 0) + rows
        pos_mask = same & (col != rowid)
        num_pos = jnp.sum(pos_mask.astype(i32), axis=1, keepdims=True)
        num_neg = _N - num_pos - 1
        valid = (num_pos > 0) & (num_neg >= 2)

        ub = u_ref[pl.ds(rows, _BLK), :]
        r = jnp.minimum(
            (ub * jnp.maximum(num_pos, 1).astype(f32)).astype(i32),
            jnp.maximum(num_pos - 1, 0))
        ri = rank_col_ref[pl.ds(rows, _BLK), :]
        rb = jnp.broadcast_to(rank_row, (_BLK, _N))
        adj = rb - (rb > ri).astype(i32)
        hit = pos_mask & (adj == r)
        d_ap = jnp.sum(jnp.where(hit, db, 0.0), axis=1, keepdims=True)

        negd = jnp.where(same, _INF, db)
        bits = jax.lax.bitcast_convert_type(negd, i32)
        keys = (bits & (-2048)) | col
        ohacc = jnp.zeros((_BLK, _N), jnp.bool_)
        for jj in range(_K):
            mnk = jnp.min(keys, axis=1, keepdims=True)
            oh = keys == mnk
            ohacc = ohacc | oh
            if jj < _K - 1:
                keys = jnp.where(oh, _MAXKEY, keys)

        dnn = jnp.where(cstar_row != lb, jnp.broadcast_to(min1_row, (_BLK, _N)),
                        jnp.broadcast_to(min2_row, (_BLK, _N)))
        sel = ohacc & (negd < _INF) & valid
        tt = (jnp.maximum(d_ap + _M1 - negd, 0.0)
              + jnp.maximum(d_ap + _M2 - dnn, 0.0))
        acc = acc + jnp.sum(jnp.where(sel, tt, 0.0), keepdims=True)
        cnt = cnt + jnp.sum(sel.astype(i32), keepdims=True)

        oh_c = (lb == jax.lax.broadcasted_iota(i32, (_BLK, _C), 1)).astype(bf16)
        dc = (((1,), (0,)), ((), ()))
        gath = (jax.lax.dot_general(oh_c, cen_hi, dc, preferred_element_type=f32)
                + jax.lax.dot_general(oh_c, cen_lo, dc, preferred_element_type=f32))
        diff = eb - gath
        csum = csum + jnp.sum(diff * diff, keepdims=True)
        return acc, cnt, csum

    zero = jnp.zeros((1, 1), f32)
    acc, cnt, csum = jax.lax.fori_loop(
        0, _NBLK, phase_b, (zero, jnp.zeros((1, 1), i32), zero))

    cntf = cnt.astype(f32)
    quad = jnp.where(cnt > 0, acc / jnp.maximum(cntf, 1.0), 0.0)
    out_ref[:, :] = quad + _CW * (csum / float(_N))


@jax.jit
def kernel(embeddings, labels, centers):
    labels = labels.astype(jnp.int32)
    u = jax.random.uniform(jax.random.key(42), (_N,))
    out = pl.pallas_call(
        _loss_kernel,
        out_shape=jax.ShapeDtypeStruct((1, 1), jnp.float32),
        scratch_shapes=[
            pltpu.VMEM((_N, _N), jnp.float32),
            pltpu.VMEM((1, _N), jnp.float32),
            pltpu.VMEM((1, _N), jnp.float32),
            pltpu.VMEM((1, _N), jnp.int32),
            pltpu.VMEM((_N, 1), jnp.int32),
        ],
    )(embeddings, labels.reshape(1, _N), labels.reshape(_N, 1),
      u.reshape(_N, 1), centers)
    return out[0, 0]
